# Initial kernel scaffold; baseline (speedup 1.0000x reference)
import jax
import jax.numpy as jnp
from jax.experimental import pallas as pl


def kernel(logits, y, borders):
    raise NotImplementedError("write your pallas kernel here")


# speedup vs baseline: 22.4001x; 22.4001x over previous
import functools

import jax
import jax.numpy as jnp
from jax.experimental import pallas as pl
from jax.experimental.pallas import tpu as pltpu

_NUM_BARS = 100
_ROWS = 1024


def _nll_block_kernel(logits_ref, y_ref, borders_ref, logw_ref, out_ref):
    l = logits_ref[...]
    yv = y_ref[...]
    borders = borders_ref[...]
    logw = logw_ref[...]

    cnt = jnp.sum((borders < yv).astype(jnp.int32), axis=1, keepdims=True)
    idx = jnp.clip(cnt - 1, 0, _NUM_BARS - 1)

    m = jnp.max(l, axis=1, keepdims=True)
    s = jnp.sum(jnp.exp(l - m), axis=1, keepdims=True)
    lse = m + jnp.log(s)

    col = jax.lax.broadcasted_iota(jnp.int32, l.shape, 1)
    sel = jnp.where(col == idx, l - logw, 0.0)
    g = jnp.sum(sel, axis=1, keepdims=True)

    nll = lse - g
    out_ref[...] = jnp.where(jnp.isnan(yv), 0.0, nll)


@jax.jit
def kernel(logits, y, borders):
    b, t, nbars = logits.shape
    n = b * t
    lf = logits.reshape(n, nbars)
    yf = y.reshape(n, 1)
    borders2 = borders.reshape(1, nbars + 1)
    logw = jnp.log(borders[1:] - borders[:-1]).reshape(1, nbars)

    grid = (n // _ROWS,)
    out = pl.pallas_call(
        _nll_block_kernel,
        grid=grid,
        in_specs=[
            pl.BlockSpec((_ROWS, nbars), lambda i: (i, 0)),
            pl.BlockSpec((_ROWS, 1), lambda i: (i, 0)),
            pl.BlockSpec((1, nbars + 1), lambda i: (0, 0)),
            pl.BlockSpec((1, nbars), lambda i: (0, 0)),
        ],
        out_specs=pl.BlockSpec((_ROWS, 1), lambda i: (i, 0)),
        out_shape=jax.ShapeDtypeStruct((n, 1), jnp.float32),
    )(lf, yf, borders2, logw)
    return out.reshape(b, t)


# speedup vs baseline: 28.5292x; 1.2736x over previous
import jax
import jax.numpy as jnp
from jax.experimental import pallas as pl

_NUM_BARS = 100
_RB = 8


def _nll_block_kernel(logits_ref, y_ref, borders_ref, logw_ref, out_ref):
    l = logits_ref[...]
    yv = y_ref[...]
    borders = borders_ref[...]
    logw = logw_ref[...]

    cnt = jnp.sum((borders < yv[..., None]).astype(jnp.int32), axis=2)
    idx = jnp.clip(cnt - 1, 0, _NUM_BARS - 1)

    m = jnp.max(l, axis=2)
    s = jnp.sum(jnp.exp(l - m[..., None]), axis=2)
    lse = m + jnp.log(s)

    col = jax.lax.broadcasted_iota(jnp.int32, l.shape, 2)
    sel = jnp.where(col == idx[..., None], l - logw, 0.0)
    g = jnp.sum(sel, axis=2)

    nll = lse - g
    out_ref[...] = jnp.where(jnp.isnan(yv), 0.0, nll)


@jax.jit
def kernel(logits, y, borders):
    b, t, nbars = logits.shape
    n = b * t
    nrows = n // 128
    l3 = logits.reshape(nrows, 128, nbars)
    y2 = y.reshape(nrows, 128)
    borders3 = borders.reshape(1, 1, nbars + 1)
    logw3 = jnp.log(borders[1:] - borders[:-1]).reshape(1, 1, nbars)

    grid = (nrows // _RB,)
    out = pl.pallas_call(
        _nll_block_kernel,
        grid=grid,
        in_specs=[
            pl.BlockSpec((_RB, 128, nbars), lambda i: (i, 0, 0)),
            pl.BlockSpec((_RB, 128), lambda i: (i, 0)),
            pl.BlockSpec((1, 1, nbars + 1), lambda i: (0, 0, 0)),
            pl.BlockSpec((1, 1, nbars), lambda i: (0, 0, 0)),
        ],
        out_specs=pl.BlockSpec((_RB, 128), lambda i: (i, 0)),
        out_shape=jax.ShapeDtypeStruct((nrows, 128), jnp.float32),
    )(l3, y2, borders3, logw3)
    return out.reshape(b, t)
